# baseline (speedup 1.0000x reference)
import jax
import jax.numpy as jnp
from jax import lax
from jax.experimental import pallas as pl
from jax.experimental.pallas import tpu as pltpu
from jax.experimental.pallas import tpu_sc as plsc

B = 4
G = 32
NV = B * G * G * G
D = 64
NRES = 2048
KJ = 16384
KF = 32768
NW = 32
OWN = NV // NW
OWN_SHIFT = 12
FLUSH = 128
CHUNK = 8192
ZROW = NRES
NL = 34


def _sc_body(protD, protZL, protZR, cj_hbm, rj_hbm, cf_hbm, rf_hbm, zsrc_hbm,
             gc_hbm,
             cb0, cb1, rb0, rb1, lastj, lastf,
             wrj, wvj, wrf, wvf, wab, wbb, wvb,
             zrows, rowsA, rowsB, sem, sem_s, sem_ms):
  wid = lax.axis_index("s") * 2 + lax.axis_index("c")
  base = wid * OWN
  iota = lax.iota(jnp.int32, 16)

  pltpu.sync_copy(zsrc_hbm, zrows)
  memset_descs = [
      pltpu.async_copy(zrows, gc_hbm.at[pl.ds(base + i * FLUSH, FLUSH)],
                       sem_ms)
      for i in range(OWN // FLUSH)
  ]

  def _clr(i, _):
    z = jnp.zeros((16,), jnp.int32)
    for u in range(4):
      lastj[pl.ds(i * 64 + u * 16, 16)] = z
      lastf[pl.ds(i * 64 + u * 16, 16)] = z
    return 0
  lax.fori_loop(0, OWN // 64, _clr, 0)

  def _scan(c_hbm, r_hbm, k, tab):
    nch = k // CHUNK
    bufs = ((cb0, rb0), (cb1, rb1))

    def launch(ch):
      cb, rb = bufs[ch % 2]
      return (pltpu.async_copy(c_hbm.at[pl.ds(ch * CHUNK, CHUNK)], cb, sem),
              pltpu.async_copy(r_hbm.at[pl.ds(ch * CHUNK, CHUNK)], rb, sem))

    pend = launch(0)
    for ch in range(nch):
      for d in pend:
        d.wait()
      if ch + 1 < nch:
        pend = launch(ch + 1)
      cb, rb = bufs[ch % 2]

      def body(i, _):
        for u in range(4):
          s = pl.ds(i * 64 + u * 16, 16)
          v = cb[s]
          r = rb[s]
          own = lax.shift_right_logical(v, OWN_SHIFT) == wid
          loc = v & (OWN - 1)
          plsc.store_scatter(tab, [loc], r + 1, mask=own)
        return 0
      lax.fori_loop(0, CHUNK // 64, body, 0)

  with jax.named_scope("scan_j"):
    _scan(cj_hbm, rj_hbm, KJ, lastj)
  with jax.named_scope("scan_f"):
    _scan(cf_hbm, rf_hbm, KF, lastf)

  def _store2(ref, pos, val, m):
    plsc.store_scatter(ref, [lax.shift_right_logical(pos, 7), pos & 127],
                       val, mask=m)

  def compact(t, carry):
    (oj, vlj, alj), (of_, vlf, alf), (ob, vlb, alb, blb) = carry
    lj = lastj[pl.ds(t * 16, 16)]
    lf = lastf[pl.ds(t * 16, 16)]
    vg = base + t * 16 + iota
    pj = lj > 0
    pf = lf > 0

    def cat2(m, ra, wr2, wv2, off, vl, al):
      mi = m.astype(jnp.int32)
      pos = off + plsc.cumsum(mi) - 1
      _store2(wr2, pos, ra, m)
      _store2(wv2, pos, vg, m)
      vmax = jnp.max(jnp.where(m, vg, -1))
      amax = jnp.max(jnp.where(m & (vg == vmax), ra, -1))
      upd = vmax > vl
      return (off + jnp.sum(mi),
              jnp.where(upd, vmax, vl), jnp.where(upd, amax, al))

    oj2, vlj2, alj2 = cat2(pj & (~pf), lj - 1, wrj, wvj, oj, vlj, alj)
    of2, vlf2, alf2 = cat2((~pj) & pf, lf - 1, wrf, wvf, of_, vlf, alf)

    mb = pj & pf
    mib = mb.astype(jnp.int32)
    posb = ob + plsc.cumsum(mib) - 1
    _store2(wab, posb, lj - 1, mb)
    _store2(wbb, posb, lf - 1, mb)
    _store2(wvb, posb, vg, mb)
    vmaxb = jnp.max(jnp.where(mb, vg, -1))
    m2b = mb & (vg == vmaxb)
    amaxb = jnp.max(jnp.where(m2b, lj - 1, -1))
    bmaxb = jnp.max(jnp.where(m2b, lf - 1, -1))
    updb = vmaxb > vlb
    ob2 = ob + jnp.sum(mib)
    vlb2 = jnp.where(updb, vmaxb, vlb)
    alb2 = jnp.where(updb, amaxb, alb)
    blb2 = jnp.where(updb, bmaxb, blb)

    return ((oj2, vlj2, alj2), (of2, vlf2, alf2), (ob2, vlb2, alb2, blb2))

  with jax.named_scope("compact"):
    (nj, vlj, alj), (nf, vlf, alf), (nb, vlb, alb, blb) = lax.fori_loop(
        0, OWN // 16, compact,
        ((0, -1, -1), (0, -1, -1), (0, -1, -1, -1)))

  with jax.named_scope("memset_drain"):
    for d in memset_descs:
      d.wait()

  def _pad(n, npad, entries):
    def pad(c, _):
      p = c * 16 + iota
      pm = (p >= n) & (p < npad)
      for ref, val in entries:
        _store2(ref, p, jnp.full((16,), val, jnp.int32), pm)
      return 0
    lax.fori_loop(lax.shift_right_logical(n, 4),
                  lax.shift_right_logical(npad + 15, 4), pad, 0)

  def _flush_pipe(n, srcA, wr2, wv2, vl, al):
    npad = ((n + 2 * FLUSH - 1) // (2 * FLUSH)) * (2 * FLUSH)
    _pad(n, npad, ((wr2, al), (wv2, vl)))

    def drain2():
      pltpu.make_async_copy(rowsA, gc_hbm.at[wv2.at[0]], sem_s).wait()
      pltpu.make_async_copy(rowsB, gc_hbm.at[wv2.at[0]], sem_s).wait()

    def flush2(f2, _):
      @pl.when(f2 > 0)
      def _():
        drain2()
      ga = pltpu.async_copy(srcA.at[wr2.at[2 * f2]], rowsA, sem)
      gb = pltpu.async_copy(srcA.at[wr2.at[2 * f2 + 1]], rowsB, sem)
      ga.wait()
      pltpu.async_copy(rowsA, gc_hbm.at[wv2.at[2 * f2]], sem_s)
      gb.wait()
      pltpu.async_copy(rowsB, gc_hbm.at[wv2.at[2 * f2 + 1]], sem_s)
      return 0
    trips2 = npad // (2 * FLUSH)
    lax.fori_loop(0, trips2, flush2, 0)

    @pl.when(trips2 > 0)
    def _():
      drain2()

  with jax.named_scope("flush_j"):
    _flush_pipe(nj, protD, wrj, wvj, vlj, alj)
  with jax.named_scope("flush_f"):
    _flush_pipe(nf, protZL, wrf, wvf, vlf, alf)

  npadb = ((nb + FLUSH - 1) // FLUSH) * FLUSH
  _pad(nb, npadb, ((wab, alb), (wbb, blb), (wvb, vlb)))

  def flushb(f, _):
    ga = pltpu.async_copy(protZR.at[wab.at[f]], rowsA, sem)
    gb = pltpu.async_copy(protZL.at[wbb.at[f]], rowsB, sem)
    ga.wait()
    gb.wait()

    def addrow(i, _):
      r = i // (D * 2 // 16)
      c = (i % (D * 2 // 16)) * 16
      rowsA[r, pl.ds(c, 16)] = rowsA[r, pl.ds(c, 16)] + rowsB[r, pl.ds(c, 16)]
      return 0
    lax.fori_loop(0, FLUSH * (D * 2 // 16), addrow, 0)
    pltpu.async_copy(rowsA, gc_hbm.at[wvb.at[f]], sem_s).wait()
    return 0
  lax.fori_loop(0, npadb // FLUSH, flushb, 0)


@jax.jit
def _sc_scatter(protD, protZL, protZR, cj, rj, cf, rf):
  mesh = plsc.VectorSubcoreMesh(core_axis_name="c", subcore_axis_name="s",
                                num_cores=2, num_subcores=16)
  zsrc = jnp.zeros((FLUSH, 2 * D), jnp.float32)
  f = pl.kernel(
      _sc_body,
      out_type=jax.ShapeDtypeStruct((NV, 2 * D), jnp.float32),
      mesh=mesh,
      compiler_params=pltpu.CompilerParams(needs_layout_passes=False),
      scratch_types=[
          pltpu.VMEM((CHUNK,), jnp.int32),
          pltpu.VMEM((CHUNK,), jnp.int32),
          pltpu.VMEM((CHUNK,), jnp.int32),
          pltpu.VMEM((CHUNK,), jnp.int32),
          pltpu.VMEM((OWN,), jnp.int32),
          pltpu.VMEM((OWN,), jnp.int32),
          pltpu.VMEM((NL, FLUSH), jnp.int32),
          pltpu.VMEM((NL, FLUSH), jnp.int32),
          pltpu.VMEM((NL, FLUSH), jnp.int32),
          pltpu.VMEM((NL, FLUSH), jnp.int32),
          pltpu.VMEM((NL, FLUSH), jnp.int32),
          pltpu.VMEM((NL, FLUSH), jnp.int32),
          pltpu.VMEM((NL, FLUSH), jnp.int32),
          pltpu.VMEM((FLUSH, 2 * D), jnp.float32),
          pltpu.VMEM((FLUSH, 2 * D), jnp.float32),
          pltpu.VMEM((FLUSH, 2 * D), jnp.float32),
          pltpu.SemaphoreType.DMA,
          pltpu.SemaphoreType.DMA,
          pltpu.SemaphoreType.DMA,
      ],
  )
  return f(protD, protZL, protZR, cj, rj, cf, rf, zsrc)


TYZ = 512
XB = 8


def _tc_body(cd, gc, oj, of):
  z = jnp.zeros((XB, TYZ, 1), jnp.float32)
  o = jnp.ones((XB, TYZ, 1), jnp.float32)
  g = gc[0]
  c = cd[0]
  catj = jnp.concatenate([c[:, :, :84], g[:, :, :D], z, o], axis=-1)
  catf = jnp.concatenate([c[:, :, 84:], g[:, :, D:], o, z], axis=-1)
  oj[0] = jnp.transpose(catj, (1, 0, 2))
  of[0] = jnp.transpose(catf, (1, 0, 2))


@jax.jit
def _tc_fuse(cd, gc):
  YZ = G * G
  nc = 150

  def in_spec(c):
    return pl.BlockSpec((1, XB, TYZ, c), lambda b, x, t: (b, x, t, 0))

  out_spec = pl.BlockSpec((1, TYZ, XB, nc), lambda b, x, t: (b, t, x, 0))
  return pl.pallas_call(
      _tc_body,
      grid=(B, G // XB, YZ // TYZ),
      in_specs=[in_spec(168), in_spec(2 * D)],
      out_specs=[out_spec, out_spec],
      out_shape=[jax.ShapeDtypeStruct((B, YZ, G, nc), jnp.float32),
                 jax.ShapeDtypeStruct((B, YZ, G, nc), jnp.float32)],
  )(cd, gc)


def kernel(voxels_allAtom_jigsaw, voxels_perAA_jigsaw, voxels_allAtom_full,
           voxels_perAA_full, prot_feats, centerIdx_jigsaw, resIds_jigsaw,
           centerIdx_full, resIds_full):
  z = jnp.zeros((NRES + 1, D), jnp.float32)
  pp = z.at[:NRES].set(prot_feats)
  protD = jnp.concatenate([pp, pp], axis=1)
  protZL = jnp.concatenate([jnp.zeros_like(pp), pp], axis=1)
  protZR = jnp.concatenate([pp, jnp.zeros_like(pp)], axis=1)
  gc = _sc_scatter(protD, protZL, protZR,
                   centerIdx_jigsaw.astype(jnp.int32),
                   resIds_jigsaw.astype(jnp.int32),
                   centerIdx_full.astype(jnp.int32),
                   resIds_full.astype(jnp.int32))
  gc4 = gc.reshape(B, G, G * G, 2 * D)
  cd = jnp.concatenate(
      [voxels_allAtom_jigsaw, voxels_perAA_jigsaw,
       voxels_allAtom_full, voxels_perAA_full],
      axis=-1).reshape(B, G, G * G, 168)
  oj, of = _tc_fuse(cd, gc4)
  oj = jnp.transpose(oj.reshape(B, G, G, G, 150), (0, 4, 1, 2, 3))
  of = jnp.transpose(of.reshape(B, G, G, G, 150), (0, 4, 1, 2, 3))
  return (oj, of)


# speedup vs baseline: 1.1885x; 1.1885x over previous
import jax
import jax.numpy as jnp
from jax import lax
from jax.experimental import pallas as pl
from jax.experimental.pallas import tpu as pltpu
from jax.experimental.pallas import tpu_sc as plsc

B = 4
G = 32
NV = B * G * G * G
D = 64
NRES = 2048
KJ = 16384
KF = 32768
NW = 32
OWN = NV // NW
OWN_SHIFT = 12
FLUSH = 128
NBUF = 4
CHUNK = 4096
NR1 = NRES + 1
NL = 34


def _sc_body(protA, cj_hbm, rj_hbm, cf_hbm, rf_hbm, zsrc_hbm,
             gc_hbm,
             cb0, cb1, rb0, rb1, lastj, lastf,
             wr, wv, wab, wbb, wvb, rows,
             sem, sem_ms, sg0, sg1, sg2, sg3, ss0, ss1, ss2, ss3):
  wid = lax.axis_index("s") * 2 + lax.axis_index("c")
  base = wid * OWN
  iota = lax.iota(jnp.int32, 16)
  sgs = (sg0, sg1, sg2, sg3)
  sss = (ss0, ss1, ss2, ss3)

  pltpu.sync_copy(zsrc_hbm, rows.at[0])
  memset_descs = [
      pltpu.async_copy(rows.at[0], gc_hbm.at[pl.ds(base + i * FLUSH, FLUSH)],
                       sem_ms)
      for i in range(OWN // FLUSH)
  ]

  def _clr(i, _):
    z = jnp.zeros((16,), jnp.int32)
    for u in range(4):
      lastj[pl.ds(i * 64 + u * 16, 16)] = z
      lastf[pl.ds(i * 64 + u * 16, 16)] = z
    return 0
  lax.fori_loop(0, OWN // 64, _clr, 0)

  def _scan(c_hbm, r_hbm, k, tab):
    nch = k // CHUNK
    bufs = ((cb0, rb0), (cb1, rb1))

    def launch(ch):
      cb, rb = bufs[ch % 2]
      return (pltpu.async_copy(c_hbm.at[pl.ds(ch * CHUNK, CHUNK)], cb, sem),
              pltpu.async_copy(r_hbm.at[pl.ds(ch * CHUNK, CHUNK)], rb, sem))

    pend = launch(0)
    for ch in range(nch):
      for d in pend:
        d.wait()
      if ch + 1 < nch:
        pend = launch(ch + 1)
      cb, rb = bufs[ch % 2]

      def body(i, _):
        for u in range(4):
          s = pl.ds(i * 64 + u * 16, 16)
          v = cb[s]
          r = rb[s]
          own = lax.shift_right_logical(v, OWN_SHIFT) == wid
          loc = v & (OWN - 1)
          plsc.store_scatter(tab, [loc], r + 1, mask=own)
        return 0
      lax.fori_loop(0, CHUNK // 64, body, 0)

  with jax.named_scope("scan_j"):
    _scan(cj_hbm, rj_hbm, KJ, lastj)
  with jax.named_scope("scan_f"):
    _scan(cf_hbm, rf_hbm, KF, lastf)

  def _store2(ref, pos, val, m):
    plsc.store_scatter(ref, [lax.shift_right_logical(pos, 7), pos & 127],
                       val, mask=m)

  def compact(t, carry):
    (os_, vls, als), (ob, vlb, alb, blb) = carry
    lj = lastj[pl.ds(t * 16, 16)]
    lf = lastf[pl.ds(t * 16, 16)]
    vg = base + t * 16 + iota
    pj = lj > 0
    pf = lf > 0

    ms = pj ^ pf
    ra = jnp.where(pj, lj - 1, lf - 1 + NR1)
    mis = ms.astype(jnp.int32)
    pos = os_ + plsc.cumsum(mis) - 1
    _store2(wr, pos, ra, ms)
    _store2(wv, pos, vg, ms)
    vmax = jnp.max(jnp.where(ms, vg, -1))
    amax = jnp.max(jnp.where(ms & (vg == vmax), ra, -1))
    upd = vmax > vls
    os2 = os_ + jnp.sum(mis)
    vls2 = jnp.where(upd, vmax, vls)
    als2 = jnp.where(upd, amax, als)

    mb = pj & pf
    mib = mb.astype(jnp.int32)
    posb = ob + plsc.cumsum(mib) - 1
    _store2(wab, posb, lj - 1 + 2 * NR1, mb)
    _store2(wbb, posb, lf - 1 + NR1, mb)
    _store2(wvb, posb, vg, mb)
    vmaxb = jnp.max(jnp.where(mb, vg, -1))
    m2b = mb & (vg == vmaxb)
    amaxb = jnp.max(jnp.where(m2b, lj - 1 + 2 * NR1, -1))
    bmaxb = jnp.max(jnp.where(m2b, lf - 1 + NR1, -1))
    updb = vmaxb > vlb
    ob2 = ob + jnp.sum(mib)
    vlb2 = jnp.where(updb, vmaxb, vlb)
    alb2 = jnp.where(updb, amaxb, alb)
    blb2 = jnp.where(updb, bmaxb, blb)

    return ((os2, vls2, als2), (ob2, vlb2, alb2, blb2))

  with jax.named_scope("compact"):
    (ns, vls, als), (nb, vlb, alb, blb) = lax.fori_loop(
        0, OWN // 16, compact, ((0, -1, -1), (0, -1, -1, -1)))

  with jax.named_scope("memset_drain"):
    for d in memset_descs:
      d.wait()

  def _pad(n, npad, entries):
    def pad(c, _):
      p = c * 16 + iota
      pm = (p >= n) & (p < npad)
      for ref, val in entries:
        _store2(ref, p, jnp.full((16,), val, jnp.int32), pm)
      return 0
    lax.fori_loop(lax.shift_right_logical(n, 4),
                  lax.shift_right_logical(npad + 15, 4), pad, 0)

  with jax.named_scope("flush_s"):
    npad = ((ns + NBUF * FLUSH - 1) // (NBUF * FLUSH)) * (NBUF * FLUSH)
    _pad(ns, npad, ((wr, als), (wv, vls)))

    def flushN(f4, _):
      for k in range(NBUF):
        @pl.when(f4 > 0)
        def _(k=k):
          pltpu.make_async_copy(rows.at[k], gc_hbm.at[wv.at[0]],
                                sss[k]).wait()
        pltpu.async_copy(protA.at[wr.at[NBUF * f4 + k]], rows.at[k], sgs[k])
      for k in range(NBUF):
        pltpu.make_async_copy(protA.at[wr.at[0]], rows.at[k], sgs[k]).wait()
        pltpu.async_copy(rows.at[k], gc_hbm.at[wv.at[NBUF * f4 + k]], sss[k])
      return 0
    tripsN = npad // (NBUF * FLUSH)
    lax.fori_loop(0, tripsN, flushN, 0)

    @pl.when(tripsN > 0)
    def _():
      for k in range(NBUF):
        pltpu.make_async_copy(rows.at[k], gc_hbm.at[wv.at[0]], sss[k]).wait()

  with jax.named_scope("flush_b"):
    npadb = ((nb + FLUSH - 1) // FLUSH) * FLUSH
    _pad(nb, npadb, ((wab, alb), (wbb, blb), (wvb, vlb)))

    def flushb(f, _):
      ga = pltpu.async_copy(protA.at[wab.at[f]], rows.at[0], sg0)
      gb = pltpu.async_copy(protA.at[wbb.at[f]], rows.at[1], sg1)
      ga.wait()
      gb.wait()

      def addrow(i, _):
        r = i // (D * 2 // 16)
        c = (i % (D * 2 // 16)) * 16
        rows[0, r, pl.ds(c, 16)] = (rows[0, r, pl.ds(c, 16)] +
                                    rows[1, r, pl.ds(c, 16)])
        return 0
      lax.fori_loop(0, FLUSH * (D * 2 // 16), addrow, 0)
      pltpu.async_copy(rows.at[0], gc_hbm.at[wvb.at[f]], ss0).wait()
      return 0
    lax.fori_loop(0, npadb // FLUSH, flushb, 0)


@jax.jit
def _sc_scatter(protA, cj, rj, cf, rf):
  mesh = plsc.VectorSubcoreMesh(core_axis_name="c", subcore_axis_name="s",
                                num_cores=2, num_subcores=16)
  zsrc = jnp.zeros((FLUSH, 2 * D), jnp.float32)
  f = pl.kernel(
      _sc_body,
      out_type=jax.ShapeDtypeStruct((NV, 2 * D), jnp.float32),
      mesh=mesh,
      compiler_params=pltpu.CompilerParams(needs_layout_passes=False),
      scratch_types=[
          pltpu.VMEM((CHUNK,), jnp.int32),
          pltpu.VMEM((CHUNK,), jnp.int32),
          pltpu.VMEM((CHUNK,), jnp.int32),
          pltpu.VMEM((CHUNK,), jnp.int32),
          pltpu.VMEM((OWN,), jnp.int32),
          pltpu.VMEM((OWN,), jnp.int32),
          pltpu.VMEM((NL, FLUSH), jnp.int32),
          pltpu.VMEM((NL, FLUSH), jnp.int32),
          pltpu.VMEM((NL, FLUSH), jnp.int32),
          pltpu.VMEM((NL, FLUSH), jnp.int32),
          pltpu.VMEM((NL, FLUSH), jnp.int32),
          pltpu.VMEM((NBUF, FLUSH, 2 * D), jnp.float32),
          pltpu.SemaphoreType.DMA,
          pltpu.SemaphoreType.DMA,
          pltpu.SemaphoreType.DMA,
          pltpu.SemaphoreType.DMA,
          pltpu.SemaphoreType.DMA,
          pltpu.SemaphoreType.DMA,
          pltpu.SemaphoreType.DMA,
          pltpu.SemaphoreType.DMA,
          pltpu.SemaphoreType.DMA,
          pltpu.SemaphoreType.DMA,
      ],
  )
  return f(protA, cj, rj, cf, rf, zsrc)


TYZ = 512
XB = 8


def _tc_body(aj, pj, af, pf, gc, oj, of):
  z = jnp.zeros((XB, TYZ, 1), jnp.float32)
  o = jnp.ones((XB, TYZ, 1), jnp.float32)
  g = gc[0]
  catj = jnp.concatenate([aj[0], pj[0], g[:, :, :D], z, o], axis=-1)
  catf = jnp.concatenate([af[0], pf[0], g[:, :, D:], o, z], axis=-1)
  oj[0] = jnp.transpose(catj, (1, 0, 2))
  of[0] = jnp.transpose(catf, (1, 0, 2))


@jax.jit
def _tc_fuse(aj, pj, af, pf, gc):
  YZ = G * G
  nc = 150

  def in_spec(c):
    return pl.BlockSpec((1, XB, TYZ, c), lambda b, x, t: (b, x, t, 0))

  out_spec = pl.BlockSpec((1, TYZ, XB, nc), lambda b, x, t: (b, t, x, 0))
  return pl.pallas_call(
      _tc_body,
      grid=(B, G // XB, YZ // TYZ),
      in_specs=[in_spec(4), in_spec(80), in_spec(4), in_spec(80),
                in_spec(2 * D)],
      out_specs=[out_spec, out_spec],
      out_shape=[jax.ShapeDtypeStruct((B, YZ, G, nc), jnp.float32),
                 jax.ShapeDtypeStruct((B, YZ, G, nc), jnp.float32)],
  )(aj, pj, af, pf, gc)


def kernel(voxels_allAtom_jigsaw, voxels_perAA_jigsaw, voxels_allAtom_full,
           voxels_perAA_full, prot_feats, centerIdx_jigsaw, resIds_jigsaw,
           centerIdx_full, resIds_full):
  z = jnp.zeros((NR1, D), jnp.float32)
  pp = z.at[:NRES].set(prot_feats)
  protA = jnp.concatenate([
      jnp.concatenate([pp, pp], axis=1),
      jnp.concatenate([jnp.zeros_like(pp), pp], axis=1),
      jnp.concatenate([pp, jnp.zeros_like(pp)], axis=1),
  ], axis=0)
  gc = _sc_scatter(protA,
                   centerIdx_jigsaw.astype(jnp.int32),
                   resIds_jigsaw.astype(jnp.int32),
                   centerIdx_full.astype(jnp.int32),
                   resIds_full.astype(jnp.int32))
  gc4 = gc.reshape(B, G, G * G, 2 * D)
  aj = voxels_allAtom_jigsaw.reshape(B, G, G * G, 4)
  pj = voxels_perAA_jigsaw.reshape(B, G, G * G, 80)
  af = voxels_allAtom_full.reshape(B, G, G * G, 4)
  pf = voxels_perAA_full.reshape(B, G, G * G, 80)
  oj, of = _tc_fuse(aj, pj, af, pf, gc4)
  oj = jnp.transpose(oj.reshape(B, G, G, G, 150), (0, 4, 1, 2, 3))
  of = jnp.transpose(of.reshape(B, G, G, G, 150), (0, 4, 1, 2, 3))
  return (oj, of)


# speedup vs baseline: 1.3730x; 1.1553x over previous
import jax
import jax.numpy as jnp
from jax import lax
from jax.experimental import pallas as pl
from jax.experimental.pallas import tpu as pltpu
from jax.experimental.pallas import tpu_sc as plsc

B = 4
G = 32
NV = B * G * G * G
D = 64
NRES = 2048
KJ = 16384
KF = 32768
NW = 32
OWN = NV // NW
OWN_SHIFT = 12
FLUSH = 128
NBUF = 4
CHUNK = 4096
NR1 = NRES + 1
NL = 34


def _sc_body(protA, cj_hbm, rj_hbm, cf_hbm, rf_hbm, zsrc_hbm,
             gc_hbm,
             cb0, cb1, rb0, rb1, lastj, lastf,
             wr, wv, wab, wbb, wvb, rows,
             sem, sem_ms, sg0, sg1, sg2, sg3, ss0, ss1, ss2, ss3):
  wid = lax.axis_index("s") * 2 + lax.axis_index("c")
  base = wid * OWN
  iota = lax.iota(jnp.int32, 16)
  sgs = (sg0, sg1, sg2, sg3)
  sss = (ss0, ss1, ss2, ss3)

  pltpu.sync_copy(zsrc_hbm, rows.at[0])
  memset_descs = [
      pltpu.async_copy(rows.at[0], gc_hbm.at[pl.ds(base + i * FLUSH, FLUSH)],
                       sem_ms)
      for i in range(OWN // FLUSH)
  ]

  def _clr(i, _):
    z = jnp.zeros((16,), jnp.int32)
    for u in range(4):
      lastj[pl.ds(i * 64 + u * 16, 16)] = z
      lastf[pl.ds(i * 64 + u * 16, 16)] = z
    return 0
  lax.fori_loop(0, OWN // 64, _clr, 0)

  def _scan(c_hbm, r_hbm, k, tab):
    nch = k // CHUNK
    bufs = ((cb0, rb0), (cb1, rb1))

    def launch(ch):
      cb, rb = bufs[ch % 2]
      return (pltpu.async_copy(c_hbm.at[pl.ds(ch * CHUNK, CHUNK)], cb, sem),
              pltpu.async_copy(r_hbm.at[pl.ds(ch * CHUNK, CHUNK)], rb, sem))

    pend = launch(0)
    for ch in range(nch):
      for d in pend:
        d.wait()
      if ch + 1 < nch:
        pend = launch(ch + 1)
      cb, rb = bufs[ch % 2]

      def body(i, _):
        for u in range(4):
          s = pl.ds(i * 64 + u * 16, 16)
          v = cb[s]
          r = rb[s]
          own = lax.shift_right_logical(v, OWN_SHIFT) == wid
          loc = v & (OWN - 1)
          plsc.store_scatter(tab, [loc], r + 1, mask=own)
        return 0
      lax.fori_loop(0, CHUNK // 64, body, 0)

  with jax.named_scope("scan_j"):
    _scan(cj_hbm, rj_hbm, KJ, lastj)
  with jax.named_scope("scan_f"):
    _scan(cf_hbm, rf_hbm, KF, lastf)

  def _store2(ref, pos, val, m):
    plsc.store_scatter(ref, [lax.shift_right_logical(pos, 7), pos & 127],
                       val, mask=m)

  def compact(t, carry):
    (os_, vls, als), (ob, vlb, alb, blb) = carry
    lj = lastj[pl.ds(t * 16, 16)]
    lf = lastf[pl.ds(t * 16, 16)]
    vg = base + t * 16 + iota
    pj = lj > 0
    pf = lf > 0

    ms = pj ^ pf
    ra = jnp.where(pj, lj - 1, lf - 1 + NR1)
    mis = ms.astype(jnp.int32)
    pos = os_ + plsc.cumsum(mis) - 1
    _store2(wr, pos, ra, ms)
    _store2(wv, pos, vg, ms)
    vmax = jnp.max(jnp.where(ms, vg, -1))
    amax = jnp.max(jnp.where(ms & (vg == vmax), ra, -1))
    upd = vmax > vls
    os2 = os_ + jnp.sum(mis)
    vls2 = jnp.where(upd, vmax, vls)
    als2 = jnp.where(upd, amax, als)

    mb = pj & pf
    mib = mb.astype(jnp.int32)
    posb = ob + plsc.cumsum(mib) - 1
    _store2(wab, posb, lj - 1 + 2 * NR1, mb)
    _store2(wbb, posb, lf - 1 + NR1, mb)
    _store2(wvb, posb, vg, mb)
    vmaxb = jnp.max(jnp.where(mb, vg, -1))
    m2b = mb & (vg == vmaxb)
    amaxb = jnp.max(jnp.where(m2b, lj - 1 + 2 * NR1, -1))
    bmaxb = jnp.max(jnp.where(m2b, lf - 1 + NR1, -1))
    updb = vmaxb > vlb
    ob2 = ob + jnp.sum(mib)
    vlb2 = jnp.where(updb, vmaxb, vlb)
    alb2 = jnp.where(updb, amaxb, alb)
    blb2 = jnp.where(updb, bmaxb, blb)

    return ((os2, vls2, als2), (ob2, vlb2, alb2, blb2))

  with jax.named_scope("compact"):
    (ns, vls, als), (nb, vlb, alb, blb) = lax.fori_loop(
        0, OWN // 16, compact, ((0, -1, -1), (0, -1, -1, -1)))

  with jax.named_scope("memset_drain"):
    for d in memset_descs:
      d.wait()

  def _pad(n, npad, entries):
    def pad(c, _):
      p = c * 16 + iota
      pm = (p >= n) & (p < npad)
      for ref, val in entries:
        _store2(ref, p, jnp.full((16,), val, jnp.int32), pm)
      return 0
    lax.fori_loop(lax.shift_right_logical(n, 4),
                  lax.shift_right_logical(npad + 15, 4), pad, 0)

  with jax.named_scope("flush_s"):
    npad = ((ns + NBUF * FLUSH - 1) // (NBUF * FLUSH)) * (NBUF * FLUSH)
    _pad(ns, npad, ((wr, als), (wv, vls)))

    def flushN(f4, _):
      for k in range(NBUF):
        @pl.when(f4 > 0)
        def _(k=k):
          pltpu.make_async_copy(rows.at[k], gc_hbm.at[wv.at[0]],
                                sss[k]).wait()
        pltpu.async_copy(protA.at[wr.at[NBUF * f4 + k]], rows.at[k], sgs[k])
      for k in range(NBUF):
        pltpu.make_async_copy(protA.at[wr.at[0]], rows.at[k], sgs[k]).wait()
        pltpu.async_copy(rows.at[k], gc_hbm.at[wv.at[NBUF * f4 + k]], sss[k])
      return 0
    tripsN = npad // (NBUF * FLUSH)
    lax.fori_loop(0, tripsN, flushN, 0)

    @pl.when(tripsN > 0)
    def _():
      for k in range(NBUF):
        pltpu.make_async_copy(rows.at[k], gc_hbm.at[wv.at[0]], sss[k]).wait()

  with jax.named_scope("flush_b"):
    npadb = ((nb + FLUSH - 1) // FLUSH) * FLUSH
    _pad(nb, npadb, ((wab, alb), (wbb, blb), (wvb, vlb)))

    def flushb(f, _):
      ga = pltpu.async_copy(protA.at[wab.at[f]], rows.at[0], sg0)
      gb = pltpu.async_copy(protA.at[wbb.at[f]], rows.at[1], sg1)
      ga.wait()
      gb.wait()

      def addrow(i, _):
        r = i // (D * 2 // 16)
        c = (i % (D * 2 // 16)) * 16
        rows[0, r, pl.ds(c, 16)] = (rows[0, r, pl.ds(c, 16)] +
                                    rows[1, r, pl.ds(c, 16)])
        return 0
      lax.fori_loop(0, FLUSH * (D * 2 // 16), addrow, 0)
      pltpu.async_copy(rows.at[0], gc_hbm.at[wvb.at[f]], ss0).wait()
      return 0
    lax.fori_loop(0, npadb // FLUSH, flushb, 0)


@jax.jit
def _sc_scatter(protA, cj, rj, cf, rf):
  mesh = plsc.VectorSubcoreMesh(core_axis_name="c", subcore_axis_name="s",
                                num_cores=2, num_subcores=16)
  zsrc = jnp.zeros((FLUSH, 2 * D), jnp.float32)
  f = pl.kernel(
      _sc_body,
      out_type=jax.ShapeDtypeStruct((NV, 2 * D), jnp.float32),
      mesh=mesh,
      compiler_params=pltpu.CompilerParams(needs_layout_passes=False),
      scratch_types=[
          pltpu.VMEM((CHUNK,), jnp.int32),
          pltpu.VMEM((CHUNK,), jnp.int32),
          pltpu.VMEM((CHUNK,), jnp.int32),
          pltpu.VMEM((CHUNK,), jnp.int32),
          pltpu.VMEM((OWN,), jnp.int32),
          pltpu.VMEM((OWN,), jnp.int32),
          pltpu.VMEM((NL, FLUSH), jnp.int32),
          pltpu.VMEM((NL, FLUSH), jnp.int32),
          pltpu.VMEM((NL, FLUSH), jnp.int32),
          pltpu.VMEM((NL, FLUSH), jnp.int32),
          pltpu.VMEM((NL, FLUSH), jnp.int32),
          pltpu.VMEM((NBUF, FLUSH, 2 * D), jnp.float32),
          pltpu.SemaphoreType.DMA,
          pltpu.SemaphoreType.DMA,
          pltpu.SemaphoreType.DMA,
          pltpu.SemaphoreType.DMA,
          pltpu.SemaphoreType.DMA,
          pltpu.SemaphoreType.DMA,
          pltpu.SemaphoreType.DMA,
          pltpu.SemaphoreType.DMA,
          pltpu.SemaphoreType.DMA,
          pltpu.SemaphoreType.DMA,
      ],
  )
  return f(protA, cj, rj, cf, rf, zsrc)


TYZ = 512
XB = 8


def _tc_body(aj, pj, af, pf, gc, oj, of):
  z = jnp.zeros((XB, TYZ, 1), jnp.float32)
  o = jnp.ones((XB, TYZ, 1), jnp.float32)
  g = gc[0]

  def unz(a):
    return jnp.transpose(a, (0, 1, 3, 2)).reshape(XB, TYZ, 4)

  catj = jnp.concatenate([unz(aj[0]), pj[0], g[:, :, :D], z, o], axis=-1)
  catf = jnp.concatenate([unz(af[0]), pf[0], g[:, :, D:], o, z], axis=-1)
  oj[0] = jnp.transpose(catj, (1, 0, 2))
  of[0] = jnp.transpose(catf, (1, 0, 2))


@jax.jit
def _tc_fuse(aj, pj, af, pf, gc):
  YZ = G * G
  nc = 150

  def in_spec(c):
    return pl.BlockSpec((1, XB, TYZ, c), lambda b, x, t: (b, x, t, 0))

  a_spec = pl.BlockSpec((1, XB, TYZ // G, 4, G),
                        lambda b, x, t: (b, x, t, 0, 0))
  out_spec = pl.BlockSpec((1, TYZ, XB, nc), lambda b, x, t: (b, t, x, 0))
  return pl.pallas_call(
      _tc_body,
      grid=(B, G // XB, YZ // TYZ),
      in_specs=[a_spec, in_spec(80), a_spec, in_spec(80),
                in_spec(2 * D)],
      out_specs=[out_spec, out_spec],
      out_shape=[jax.ShapeDtypeStruct((B, YZ, G, nc), jnp.float32),
                 jax.ShapeDtypeStruct((B, YZ, G, nc), jnp.float32)],
  )(aj, pj, af, pf, gc)


def kernel(voxels_allAtom_jigsaw, voxels_perAA_jigsaw, voxels_allAtom_full,
           voxels_perAA_full, prot_feats, centerIdx_jigsaw, resIds_jigsaw,
           centerIdx_full, resIds_full):
  z = jnp.zeros((NR1, D), jnp.float32)
  pp = z.at[:NRES].set(prot_feats)
  protA = jnp.concatenate([
      jnp.concatenate([pp, pp], axis=1),
      jnp.concatenate([jnp.zeros_like(pp), pp], axis=1),
      jnp.concatenate([pp, jnp.zeros_like(pp)], axis=1),
  ], axis=0)
  gc = _sc_scatter(protA,
                   centerIdx_jigsaw.astype(jnp.int32),
                   resIds_jigsaw.astype(jnp.int32),
                   centerIdx_full.astype(jnp.int32),
                   resIds_full.astype(jnp.int32))
  gc4 = gc.reshape(B, G, G * G, 2 * D)
  aj = jnp.transpose(voxels_allAtom_jigsaw, (0, 1, 2, 4, 3))
  af = jnp.transpose(voxels_allAtom_full, (0, 1, 2, 4, 3))
  pj = voxels_perAA_jigsaw.reshape(B, G, G * G, 80)
  pf = voxels_perAA_full.reshape(B, G, G * G, 80)
  oj, of = _tc_fuse(aj, pj, af, pf, gc4)
  oj = jnp.transpose(oj.reshape(B, G, G, G, 150), (0, 4, 1, 2, 3))
  of = jnp.transpose(of.reshape(B, G, G, G, 150), (0, 4, 1, 2, 3))
  return (oj, of)
